# baseline (speedup 1.0000x reference)
import functools

import jax
import jax.numpy as jnp
from jax import lax
from jax.experimental import pallas as pl
from jax.experimental.pallas import tpu as pltpu
from jax.experimental.pallas import tpu_sc as plsc

T, B, D = 8192, 4, 1024
N = T * B
NC, NS, L = 2, 16, 16
NW = NC * NS
T_PER_W = T // NW
KT = 4
K = KT * B
NCHUNK = T_PER_W // KT
NBUF = 3


def _cumsum_body(f_ref, idx_ref):
    f = f_ref[...]
    t = lax.broadcasted_iota(jnp.int32, (T, B), 0)
    c = jnp.where(t == 0, 0, f.astype(jnp.int32))
    k = 1
    while k < T:
        z = jnp.zeros((k, B), jnp.int32)
        c = c + jnp.concatenate([z, c[:-k, :]], axis=0)
        k *= 2
    idx_ref[...] = c


def _cumsum(flags):
    return pl.pallas_call(
        _cumsum_body,
        out_shape=jax.ShapeDtypeStruct((T, B), jnp.int32),
    )(flags)


def _sc_body(idx_hbm, x_hbm, tbl_hbm, out_hbm, idx_v, *bufs):
    pe = bufs[0:NBUF]
    xb = bufs[NBUF:2 * NBUF]
    gsem = bufs[2 * NBUF:3 * NBUF]
    xsem = bufs[3 * NBUF:4 * NBUF]
    osem = bufs[4 * NBUF:5 * NBUF]
    cc = lax.axis_index("c")
    ss = lax.axis_index("s")
    wid = ss * NC + cc
    t0 = wid * T_PER_W
    base = t0 * B

    pltpu.sync_copy(idx_hbm.at[pl.ds(base, T_PER_W * B)], idx_v)

    def issue_in(g, b):
        pltpu.async_copy(tbl_hbm.at[idx_v.at[pl.ds(g * K, K)]], pe[b], gsem[b])
        pltpu.async_copy(x_hbm.at[pl.ds(t0 + g * KT, KT)], xb[b], xsem[b])

    def process(g, b, prefetch_b):
        @pl.when(g >= 1)
        def _():
            pltpu.make_async_copy(xb[prefetch_b], out_hbm.at[pl.ds(t0, KT)],
                                  osem[prefetch_b]).wait()

        @pl.when(g + NBUF - 1 < NCHUNK)
        def _():
            issue_in(g + NBUF - 1, prefetch_b)

        pltpu.make_async_copy(x_hbm.at[pl.ds(t0, KT)], pe[b], gsem[b]).wait()
        pltpu.make_async_copy(x_hbm.at[pl.ds(t0, KT)], xb[b], xsem[b]).wait()

        def add_row(r, _):
            t = r // B
            bb = lax.rem(r, B)
            for j in range(D // L):
                sl = pl.ds(j * L, L)
                plsc.addupdate(xb[b].at[t, bb, sl], pe[b][r, sl])
            return 0

        lax.fori_loop(0, K, add_row, 0)
        pltpu.async_copy(xb[b], out_hbm.at[pl.ds(t0 + g * KT, KT)], osem[b])

    for b in range(NBUF - 1):
        issue_in(b, b)

    def group(p, _):
        for b in range(NBUF):
            g = p * NBUF + b
            process(g, b, (b + NBUF - 1) % NBUF)
        return 0

    lax.fori_loop(0, NCHUNK // NBUF, group, 0)
    for g in range(NCHUNK - NCHUNK % NBUF, NCHUNK):
        process(g, g % NBUF, (g + NBUF - 1) % NBUF)
    bl = (NCHUNK - 1) % NBUF
    pltpu.make_async_copy(xb[bl], out_hbm.at[pl.ds(t0, KT)], osem[bl]).wait()


def _sc_gather_add(idx, x, tbl):
    mesh = plsc.VectorSubcoreMesh(core_axis_name="c", subcore_axis_name="s")
    fn = functools.partial(
        pl.kernel,
        mesh=mesh,
        out_type=jax.ShapeDtypeStruct((T, B, D), jnp.float32),
        scratch_types=(
            [pltpu.VMEM((T_PER_W * B,), jnp.int32)]
            + [pltpu.VMEM((K, D), jnp.float32) for _ in range(NBUF)]
            + [pltpu.VMEM((KT, B, D), jnp.float32) for _ in range(NBUF)]
            + [pltpu.SemaphoreType.DMA for _ in range(3 * NBUF)]
        ),
    )(_sc_body)
    return fn(idx, x, tbl)


def kernel(x_original, x_projected_to_d_model, pe_table):
    flags = x_original[:, :, -1]
    idx = _cumsum(flags).reshape(N)
    return _sc_gather_add(idx, x_projected_to_d_model, pe_table)
